# baseline (speedup 1.0000x reference)
import functools

import jax
import jax.numpy as jnp
from jax import lax
from jax.experimental import pallas as pl
from jax.experimental.pallas import tpu as pltpu
from jax.experimental.pallas import tpu_sc as plsc

N_NODES = 10000
N_EDGES = 320000
D = 128
CPAD = 4
CHUNK = 128
NCHUNKS = N_EDGES // CHUNK
NGRP = CHUNK // 16

NC = 2
NS = 16
NW = NC * NS

ZCH = 80
NZCH = N_NODES // ZCH


def _leaky(x):
    return jnp.where(x > 0, x, 0.2 * x)


UNROLL = 6


def _sc_gather_body(nch, p_hbm, q_hbm, cx_hbm, cy_hbm, cz_hbm, rc_hbm, wr_hbm,
                    esum_hbm,
                    idx_v, buf_p, buf_q, cx_t, cy_t, cz_t, wr_t,
                    gsem0, gsem1, wsem0, wsem1, wsem2):
    NLOOP = UNROLL * pl.cdiv(pl.cdiv(nch, NW), UNROLL)
    wid = lax.axis_index("s") * NC + lax.axis_index("c")
    gsems = (gsem0, gsem1)
    wsems = (wsem0, wsem1, wsem2)

    pltpu.sync_copy(cx_hbm, cx_t)
    pltpu.sync_copy(cy_hbm, cy_t)
    pltpu.sync_copy(cz_hbm, cz_t)
    pltpu.sync_copy(wr_hbm, wr_t)
    wr_vecs = [wr_t[pl.ds(k * 16, 16)] for k in range(D // 16)]

    def cid_of(i):
        return i * NW + wid

    def issue(qs, ps, cid):
        pltpu.sync_copy(rc_hbm.at[cid], idx_v.at[qs])
        pltpu.async_copy(p_hbm.at[idx_v.at[qs, 0]], buf_p.at[ps], gsems[qs])
        pltpu.async_copy(q_hbm.at[idx_v.at[qs, 1]], buf_q.at[qs], gsems[qs])

    def wait_gathers(qs, ps):
        pltpu.make_async_copy(p_hbm.at[idx_v.at[qs, 0]], buf_p.at[ps],
                              gsems[qs]).wait()
        pltpu.make_async_copy(q_hbm.at[idx_v.at[qs, 1]], buf_q.at[qs],
                              gsems[qs]).wait()

    def write(ps, cid):
        pltpu.async_copy(buf_p.at[ps], esum_hbm.at[pl.ds(cid * CHUNK, CHUNK)],
                         wsems[ps])

    def wait_write(ps):
        pltpu.make_async_copy(buf_p.at[ps], esum_hbm.at[pl.ds(0, CHUNK)],
                              wsems[ps]).wait()

    def fuse(qs, ps):
        def grp_body(g, carry):
            iv = idx_v[qs, 0, pl.ds(g * 16, 16)]
            jv = idx_v[qs, 1, pl.ds(g * 16, 16)]
            dx = plsc.load_gather(cx_t, [iv]) - plsc.load_gather(cx_t, [jv])
            dy = plsc.load_gather(cy_t, [iv]) - plsc.load_gather(cy_t, [jv])
            dz = plsc.load_gather(cz_t, [iv]) - plsc.load_gather(cz_t, [jv])
            rad = dx * dx + dy * dy + dz * dz
            for l in range(16):
                e = g * 16 + l
                r = rad[l]
                for k in range(D // 16):
                    sl = pl.ds(k * 16, 16)
                    plsc.addupdate(buf_p.at[ps, e, sl],
                                   buf_q[qs, e, sl] + r * wr_vecs[k])
            return carry

        lax.fori_loop(0, NGRP, grp_body, 0)

    issue(0, 0, cid_of(0))

    def outer(j, carry):
        for t in range(UNROLL):
            i = j * UNROLL + t
            qs, ps = t % 2, t % 3
            qs_n, ps_n = (t + 1) % 2, (t + 1) % 3

            @pl.when(cid_of(i + 1) < nch)
            def _():
                @pl.when(i >= 2)
                def _():
                    wait_write(ps_n)
                issue(qs_n, ps_n, cid_of(i + 1))

            @pl.when(cid_of(i) < nch)
            def _():
                wait_gathers(qs, ps)
                fuse(qs, ps)
                write(ps, cid_of(i))

        return carry

    lax.fori_loop(0, NLOOP // UNROLL, outer, 0)
    wait_write(0)
    wait_write(1)
    wait_write(2)


@functools.lru_cache(maxsize=None)
def _sc_gather(nch):
    return pl.kernel(
        functools.partial(_sc_gather_body, nch),
        out_type=jax.ShapeDtypeStruct((nch * CHUNK, D), jnp.float32),
        mesh=plsc.VectorSubcoreMesh(
            core_axis_name="c", subcore_axis_name="s",
            num_cores=NC, num_subcores=NS),
        scratch_types=[
            pltpu.VMEM((2, 2, CHUNK), jnp.int32),
            pltpu.VMEM((3, CHUNK, D), jnp.float32),
            pltpu.VMEM((2, CHUNK, D), jnp.float32),
            pltpu.VMEM((N_NODES,), jnp.float32),
            pltpu.VMEM((N_NODES,), jnp.float32),
            pltpu.VMEM((N_NODES,), jnp.float32),
            pltpu.VMEM((D,), jnp.float32),
            pltpu.SemaphoreType.DMA,
            pltpu.SemaphoreType.DMA,
            pltpu.SemaphoreType.DMA,
            pltpu.SemaphoreType.DMA,
            pltpu.SemaphoreType.DMA,
        ],
        compiler_params=pltpu.CompilerParams(needs_layout_passes=False),
    )


def _sc_scatter_body(nch, ef_hbm, rc_hbm, zero_hbm, out_hbm,
                     idx_v, ef_v, agg_sh,
                     rsem0, rsem1, ssem0, ssem1):
    NSTEPS = 2 * pl.cdiv(pl.cdiv(pl.cdiv(nch, NC), NS), 2)
    c = lax.axis_index("c")
    s = lax.axis_index("s")
    rsems = (rsem0, rsem1)
    ssems = (ssem0, ssem1)

    z_v = ef_v.at[0, pl.ds(0, ZCH)]
    pltpu.sync_copy(zero_hbm, z_v)
    for i in range(pl.cdiv(NZCH, NS)):
        j = i * NS + s

        @pl.when(j < NZCH)
        def _():
            pltpu.sync_copy(z_v, agg_sh.at[pl.ds(j * ZCH, ZCH)])

    plsc.subcore_barrier()

    def jm_of(i):
        return i * NS + s

    def issue_read(b, m):
        pltpu.async_copy(rc_hbm.at[m], idx_v.at[b], rsems[b])
        pltpu.async_copy(ef_hbm.at[pl.ds(m * CHUNK, CHUNK)], ef_v.at[b],
                         rsems[b])

    def wait_read(b):
        pltpu.make_async_copy(rc_hbm.at[0], idx_v.at[b], rsems[b]).wait()
        pltpu.make_async_copy(ef_hbm.at[pl.ds(0, CHUNK)], ef_v.at[b],
                              rsems[b]).wait()

    def issue_scatter(b):
        pltpu.async_copy(ef_v.at[b], agg_sh.at[idx_v.at[b, 0]], ssems[b],
                         add=True)

    def wait_scatter(b):
        pltpu.make_async_copy(ef_v.at[b], agg_sh.at[idx_v.at[b, 0]],
                              ssems[b]).wait()

    issue_read(0, jm_of(0) * NC + c)

    def outer(jj, carry):
        for t in range(2):
            i = jj * 2 + t
            b, nb = t, 1 - t

            @pl.when(jm_of(i + 1) * NC + c < nch)
            def _():
                @pl.when(i >= 1)
                def _():
                    wait_scatter(nb)
                issue_read(nb, jm_of(i + 1) * NC + c)

            @pl.when(jm_of(i) * NC + c < nch)
            def _():
                wait_read(b)
                issue_scatter(b)

        return carry

    lax.fori_loop(0, NSTEPS // 2, outer, 0)
    wait_scatter(0)
    wait_scatter(1)
    plsc.subcore_barrier()

    for i in range(pl.cdiv(NZCH, NS)):
        j = i * NS + s

        @pl.when(j < NZCH)
        def _():
            off = j * ZCH
            pltpu.sync_copy(agg_sh.at[pl.ds(off, ZCH)], z_v)
            pltpu.sync_copy(z_v, out_hbm.at[c, pl.ds(off, ZCH)])


@functools.lru_cache(maxsize=None)
def _sc_scatter(nch):
    return pl.kernel(
        functools.partial(_sc_scatter_body, nch),
        out_type=jax.ShapeDtypeStruct((NC, N_NODES, D), jnp.float32),
        mesh=plsc.VectorSubcoreMesh(
            core_axis_name="c", subcore_axis_name="s",
            num_cores=NC, num_subcores=NS),
        scratch_types=[
            pltpu.VMEM((2, 2, CHUNK), jnp.int32),
            pltpu.VMEM((2, CHUNK, D), jnp.float32),
            pltpu.VMEM_SHARED((N_NODES, D), jnp.float32),
            pltpu.SemaphoreType.DMA,
            pltpu.SemaphoreType.DMA,
            pltpu.SemaphoreType.DMA,
            pltpu.SemaphoreType.DMA,
        ],
    )


def _tc_prep_body(h_ref, at_ref, bt_ref, p_ref, q_ref):
    hh = h_ref[...]
    p_ref[...] = jnp.dot(hh, at_ref[...], preferred_element_type=jnp.float32)
    q_ref[...] = jnp.dot(hh, bt_ref[...], preferred_element_type=jnp.float32)


def _tc_prep(h, At, Bt):
    return pl.pallas_call(
        _tc_prep_body,
        out_shape=(
            jax.ShapeDtypeStruct((N_NODES, D), jnp.float32),
            jax.ShapeDtypeStruct((N_NODES, D), jnp.float32),
        ),
    )(h, At, Bt)


BE = 4000


def _tc_edge_body(es_ref, b1_ref, w2_ref, b2_ref, out_ref):
    x = _leaky(es_ref[...] + b1_ref[...])
    y = jnp.dot(x, w2_ref[...], preferred_element_type=jnp.float32) + b2_ref[...]
    out_ref[...] = _leaky(y)


def _tc_edge(esum, be1, W2t, be2):
    ne = esum.shape[0]
    grid = (ne // BE,)
    return pl.pallas_call(
        _tc_edge_body,
        grid=grid,
        in_specs=[
            pl.BlockSpec((BE, D), lambda i: (i, 0)),
            pl.BlockSpec((1, D), lambda i: (0, 0)),
            pl.BlockSpec((D, D), lambda i: (0, 0)),
            pl.BlockSpec((1, D), lambda i: (0, 0)),
        ],
        out_specs=pl.BlockSpec((BE, D), lambda i: (i, 0)),
        out_shape=jax.ShapeDtypeStruct((ne, D), jnp.float32),
    )(esum, be1, W2t, be2)


def _tc_node_body(h_ref, agg1_ref, agg2_ref, w1h_ref, w1a_ref, b1_ref,
                  w2_ref, b2_ref, out_ref):
    hh = h_ref[...]
    agg = (agg1_ref[0] + agg1_ref[1]) + (agg2_ref[0] + agg2_ref[1])
    x = (jnp.dot(hh, w1h_ref[...], preferred_element_type=jnp.float32)
         + jnp.dot(agg, w1a_ref[...], preferred_element_type=jnp.float32)
         + b1_ref[...])
    x = _leaky(x)
    y = jnp.dot(x, w2_ref[...], preferred_element_type=jnp.float32) + b2_ref[...]
    out_ref[...] = hh + y


def _tc_node(h, aggp1, aggp2, W1ht, W1at, bn1, W2t, bn2):
    return pl.pallas_call(
        _tc_node_body,
        out_shape=jax.ShapeDtypeStruct((N_NODES, D), jnp.float32),
    )(h, aggp1, aggp2, W1ht, W1at, bn1, W2t, bn2)


def kernel(h, edge_index, coord, We1, be1, We2, be2, Wn1, bn1, Wn2, bn2):
    f32 = jnp.float32
    row = edge_index[0].astype(jnp.int32)
    col = edge_index[1].astype(jnp.int32)
    rc = jnp.stack([row.reshape(NCHUNKS, CHUNK),
                    col.reshape(NCHUNKS, CHUNK)], axis=1)
    cx = coord[:, 0].astype(f32)
    cy = coord[:, 1].astype(f32)
    cz = coord[:, 2].astype(f32)

    At = We1[:, :D].T
    Bt = We1[:, D:2 * D].T
    wr = We1[:, 2 * D].reshape(1, D)
    b1e = be1.reshape(1, D)
    W2t = We2.T
    b2e = be2.reshape(1, D)
    W1ht = Wn1[:, :D].T
    W1at = Wn1[:, D:].T
    b1n = bn1.reshape(1, D)
    W2nt = Wn2.T
    b2n = bn2.reshape(1, D)

    P, Q = _tc_prep(h, At, Bt)
    zeros_tile = jnp.zeros((ZCH, D), f32)

    hch = NCHUNKS // 2
    wrf = wr.reshape(D)
    efs, aggs = [], []
    for hf in range(2):
        rch = lax.slice_in_dim(rc, hf * hch, (hf + 1) * hch, axis=0)
        esum_h = _sc_gather(hch)(P, Q, cx, cy, cz, rch, wrf)
        ef_h = _tc_edge(esum_h, b1e, W2t, b2e)
        aggs.append(_sc_scatter(hch)(ef_h, rch, zeros_tile))
        efs.append(ef_h)

    edge_feat = jnp.concatenate(efs, axis=0)
    h_out = _tc_node(h, aggs[0], aggs[1], W1ht, W1at, b1n, W2nt, b2n)
    return (h_out, coord, edge_feat)


# speedup vs baseline: 1.1795x; 1.1795x over previous
import functools

import jax
import jax.numpy as jnp
from jax import lax
from jax.experimental import pallas as pl
from jax.experimental.pallas import tpu as pltpu
from jax.experimental.pallas import tpu_sc as plsc

N_NODES = 10000
N_EDGES = 320000
D = 128
CPAD = 4
CHUNK = 128
NCHUNKS = N_EDGES // CHUNK
NGRP = CHUNK // 16

NC = 2
NS = 16
NW = NC * NS

ZCH = 80
NZCH = N_NODES // ZCH


def _leaky(x):
    return jnp.where(x > 0, x, 0.2 * x)


UNROLL = 6


def _sc_gather_body(nch, p_hbm, q_hbm, cx_hbm, cy_hbm, cz_hbm, rc_hbm,
                    esum_hbm, rad_hbm,
                    idx_v, buf_p, buf_q, buf_rad, cx_t, cy_t, cz_t,
                    gsem0, gsem1, wsem0, wsem1, wsem2):
    NLOOP = UNROLL * pl.cdiv(pl.cdiv(nch, NW), UNROLL)
    wid = lax.axis_index("s") * NC + lax.axis_index("c")
    gsems = (gsem0, gsem1)
    wsems = (wsem0, wsem1, wsem2)

    pltpu.sync_copy(cx_hbm, cx_t)
    pltpu.sync_copy(cy_hbm, cy_t)
    pltpu.sync_copy(cz_hbm, cz_t)

    def cid_of(i):
        return i * NW + wid

    def issue(qs, ps, cid):
        pltpu.sync_copy(rc_hbm.at[cid], idx_v.at[qs])
        pltpu.async_copy(p_hbm.at[idx_v.at[qs, 0]], buf_p.at[ps], gsems[qs])
        pltpu.async_copy(q_hbm.at[idx_v.at[qs, 1]], buf_q.at[qs], gsems[qs])

    def wait_gathers(qs, ps):
        pltpu.make_async_copy(p_hbm.at[idx_v.at[qs, 0]], buf_p.at[ps],
                              gsems[qs]).wait()
        pltpu.make_async_copy(q_hbm.at[idx_v.at[qs, 1]], buf_q.at[qs],
                              gsems[qs]).wait()

    def write(ps, cid):
        pltpu.async_copy(buf_p.at[ps], esum_hbm.at[pl.ds(cid * CHUNK, CHUNK)],
                         wsems[ps])
        pltpu.async_copy(buf_rad.at[ps], rad_hbm.at[pl.ds(cid * CHUNK, CHUNK)],
                         wsems[ps])

    def wait_write(ps):
        pltpu.make_async_copy(buf_p.at[ps], esum_hbm.at[pl.ds(0, CHUNK)],
                              wsems[ps]).wait()
        pltpu.make_async_copy(buf_rad.at[ps], rad_hbm.at[pl.ds(0, CHUNK)],
                              wsems[ps]).wait()

    def fuse(qs, ps):
        def grp_body(g, carry):
            iv = idx_v[qs, 0, pl.ds(g * 16, 16)]
            jv = idx_v[qs, 1, pl.ds(g * 16, 16)]
            dx = plsc.load_gather(cx_t, [iv]) - plsc.load_gather(cx_t, [jv])
            dy = plsc.load_gather(cy_t, [iv]) - plsc.load_gather(cy_t, [jv])
            dz = plsc.load_gather(cz_t, [iv]) - plsc.load_gather(cz_t, [jv])
            buf_rad[ps, pl.ds(g * 16, 16)] = dx * dx + dy * dy + dz * dz
            return carry

        lax.fori_loop(0, NGRP, grp_body, 0)

        def add_body(e, carry):
            for k in range(D // 16):
                sl = pl.ds(k * 16, 16)
                plsc.addupdate(buf_p.at[ps, e, sl], buf_q[qs, e, sl])
            return carry

        lax.fori_loop(0, CHUNK, add_body, 0)

    issue(0, 0, cid_of(0))

    def outer(j, carry):
        for t in range(UNROLL):
            i = j * UNROLL + t
            qs, ps = t % 2, t % 3
            qs_n, ps_n = (t + 1) % 2, (t + 1) % 3

            @pl.when(cid_of(i + 1) < nch)
            def _():
                @pl.when(i >= 2)
                def _():
                    wait_write(ps_n)
                issue(qs_n, ps_n, cid_of(i + 1))

            @pl.when(cid_of(i) < nch)
            def _():
                wait_gathers(qs, ps)
                fuse(qs, ps)
                write(ps, cid_of(i))

        return carry

    lax.fori_loop(0, NLOOP // UNROLL, outer, 0)
    wait_write(0)
    wait_write(1)
    wait_write(2)


@functools.lru_cache(maxsize=None)
def _sc_gather(nch):
    return pl.kernel(
        functools.partial(_sc_gather_body, nch),
        out_type=(
            jax.ShapeDtypeStruct((nch * CHUNK, D), jnp.float32),
            jax.ShapeDtypeStruct((nch * CHUNK,), jnp.float32),
        ),
        mesh=plsc.VectorSubcoreMesh(
            core_axis_name="c", subcore_axis_name="s",
            num_cores=NC, num_subcores=NS),
        scratch_types=[
            pltpu.VMEM((2, 2, CHUNK), jnp.int32),
            pltpu.VMEM((3, CHUNK, D), jnp.float32),
            pltpu.VMEM((2, CHUNK, D), jnp.float32),
            pltpu.VMEM((3, CHUNK), jnp.float32),
            pltpu.VMEM((N_NODES,), jnp.float32),
            pltpu.VMEM((N_NODES,), jnp.float32),
            pltpu.VMEM((N_NODES,), jnp.float32),
            pltpu.SemaphoreType.DMA,
            pltpu.SemaphoreType.DMA,
            pltpu.SemaphoreType.DMA,
            pltpu.SemaphoreType.DMA,
            pltpu.SemaphoreType.DMA,
        ],
        compiler_params=pltpu.CompilerParams(needs_layout_passes=False),
    )


def _sc_scatter_body(nch, ef_hbm, rc_hbm, zero_hbm, out_hbm,
                     idx_v, ef_v, agg_sh,
                     rsem0, rsem1, ssem0, ssem1):
    NSTEPS = 2 * pl.cdiv(pl.cdiv(pl.cdiv(nch, NC), NS), 2)
    c = lax.axis_index("c")
    s = lax.axis_index("s")
    rsems = (rsem0, rsem1)
    ssems = (ssem0, ssem1)

    z_v = ef_v.at[0, pl.ds(0, ZCH)]
    pltpu.sync_copy(zero_hbm, z_v)
    for i in range(pl.cdiv(NZCH, NS)):
        j = i * NS + s

        @pl.when(j < NZCH)
        def _():
            pltpu.sync_copy(z_v, agg_sh.at[pl.ds(j * ZCH, ZCH)])

    plsc.subcore_barrier()

    def jm_of(i):
        return i * NS + s

    def issue_read(b, m):
        pltpu.async_copy(rc_hbm.at[m], idx_v.at[b], rsems[b])
        pltpu.async_copy(ef_hbm.at[pl.ds(m * CHUNK, CHUNK)], ef_v.at[b],
                         rsems[b])

    def wait_read(b):
        pltpu.make_async_copy(rc_hbm.at[0], idx_v.at[b], rsems[b]).wait()
        pltpu.make_async_copy(ef_hbm.at[pl.ds(0, CHUNK)], ef_v.at[b],
                              rsems[b]).wait()

    def issue_scatter(b):
        pltpu.async_copy(ef_v.at[b], agg_sh.at[idx_v.at[b, 0]], ssems[b],
                         add=True)

    def wait_scatter(b):
        pltpu.make_async_copy(ef_v.at[b], agg_sh.at[idx_v.at[b, 0]],
                              ssems[b]).wait()

    issue_read(0, jm_of(0) * NC + c)

    def outer(jj, carry):
        for t in range(2):
            i = jj * 2 + t
            b, nb = t, 1 - t

            @pl.when(jm_of(i + 1) * NC + c < nch)
            def _():
                @pl.when(i >= 1)
                def _():
                    wait_scatter(nb)
                issue_read(nb, jm_of(i + 1) * NC + c)

            @pl.when(jm_of(i) * NC + c < nch)
            def _():
                wait_read(b)
                issue_scatter(b)

        return carry

    lax.fori_loop(0, NSTEPS // 2, outer, 0)
    wait_scatter(0)
    wait_scatter(1)
    plsc.subcore_barrier()

    for i in range(pl.cdiv(NZCH, NS)):
        j = i * NS + s

        @pl.when(j < NZCH)
        def _():
            off = j * ZCH
            pltpu.sync_copy(agg_sh.at[pl.ds(off, ZCH)], z_v)
            pltpu.sync_copy(z_v, out_hbm.at[c, pl.ds(off, ZCH)])


@functools.lru_cache(maxsize=None)
def _sc_scatter(nch):
    return pl.kernel(
        functools.partial(_sc_scatter_body, nch),
        out_type=jax.ShapeDtypeStruct((NC, N_NODES, D), jnp.float32),
        mesh=plsc.VectorSubcoreMesh(
            core_axis_name="c", subcore_axis_name="s",
            num_cores=NC, num_subcores=NS),
        scratch_types=[
            pltpu.VMEM((2, 2, CHUNK), jnp.int32),
            pltpu.VMEM((2, CHUNK, D), jnp.float32),
            pltpu.VMEM_SHARED((N_NODES, D), jnp.float32),
            pltpu.SemaphoreType.DMA,
            pltpu.SemaphoreType.DMA,
            pltpu.SemaphoreType.DMA,
            pltpu.SemaphoreType.DMA,
        ],
    )


def _tc_prep_body(h_ref, at_ref, bt_ref, p_ref, q_ref):
    hh = h_ref[...]
    p_ref[...] = jnp.dot(hh, at_ref[...], preferred_element_type=jnp.float32)
    q_ref[...] = jnp.dot(hh, bt_ref[...], preferred_element_type=jnp.float32)


def _tc_prep(h, At, Bt):
    return pl.pallas_call(
        _tc_prep_body,
        out_shape=(
            jax.ShapeDtypeStruct((N_NODES, D), jnp.float32),
            jax.ShapeDtypeStruct((N_NODES, D), jnp.float32),
        ),
    )(h, At, Bt)


BE = 3200


def _tc_edge_body(es_ref, rad_ref, mf_ref, ones_ref, wr_ref, b1_ref, w2_ref,
                  b2_ref, out_ref):
    radm = rad_ref[0]
    rows = jnp.broadcast_to(radm[:, None, :], (BE // CHUNK, CHUNK, CHUNK))
    rows = rows.reshape(BE, CHUNK)
    radial = jnp.dot(rows * mf_ref[...], ones_ref[...],
                     preferred_element_type=jnp.float32)
    x = _leaky(es_ref[...] + radial * wr_ref[...] + b1_ref[...])
    y = jnp.dot(x, w2_ref[...], preferred_element_type=jnp.float32) + b2_ref[...]
    out_ref[...] = _leaky(y)


def _tc_edge(esum, radp, mf, ones_col, wr, be1, W2t, be2):
    ne = esum.shape[0]
    grid = (ne // BE,)
    return pl.pallas_call(
        _tc_edge_body,
        grid=grid,
        in_specs=[
            pl.BlockSpec((BE, D), lambda i: (i, 0)),
            pl.BlockSpec((1, BE // CHUNK, CHUNK), lambda i: (i, 0, 0)),
            pl.BlockSpec((BE, CHUNK), lambda i: (0, 0)),
            pl.BlockSpec((CHUNK, 1), lambda i: (0, 0)),
            pl.BlockSpec((1, D), lambda i: (0, 0)),
            pl.BlockSpec((1, D), lambda i: (0, 0)),
            pl.BlockSpec((D, D), lambda i: (0, 0)),
            pl.BlockSpec((1, D), lambda i: (0, 0)),
        ],
        out_specs=pl.BlockSpec((BE, D), lambda i: (i, 0)),
        out_shape=jax.ShapeDtypeStruct((ne, D), jnp.float32),
    )(esum, radp, mf, ones_col, wr, be1, W2t, be2)


def _tc_node_body(h_ref, agg_ref, w1h_ref, w1a_ref, b1_ref,
                  w2_ref, b2_ref, out_ref):
    hh = h_ref[...]
    agg = agg_ref[0] + agg_ref[1]
    x = (jnp.dot(hh, w1h_ref[...], preferred_element_type=jnp.float32)
         + jnp.dot(agg, w1a_ref[...], preferred_element_type=jnp.float32)
         + b1_ref[...])
    x = _leaky(x)
    y = jnp.dot(x, w2_ref[...], preferred_element_type=jnp.float32) + b2_ref[...]
    out_ref[...] = hh + y


def _tc_node(h, aggp, W1ht, W1at, bn1, W2t, bn2):
    return pl.pallas_call(
        _tc_node_body,
        out_shape=jax.ShapeDtypeStruct((N_NODES, D), jnp.float32),
    )(h, aggp, W1ht, W1at, bn1, W2t, bn2)


def kernel(h, edge_index, coord, We1, be1, We2, be2, Wn1, bn1, Wn2, bn2):
    f32 = jnp.float32
    row = edge_index[0].astype(jnp.int32)
    col = edge_index[1].astype(jnp.int32)
    rc = jnp.stack([row.reshape(NCHUNKS, CHUNK),
                    col.reshape(NCHUNKS, CHUNK)], axis=1)
    cx = coord[:, 0].astype(f32)
    cy = coord[:, 1].astype(f32)
    cz = coord[:, 2].astype(f32)

    At = We1[:, :D].T
    Bt = We1[:, D:2 * D].T
    wr = We1[:, 2 * D].reshape(1, D)
    b1e = be1.reshape(1, D)
    W2t = We2.T
    b2e = be2.reshape(1, D)
    W1ht = Wn1[:, :D].T
    W1at = Wn1[:, D:].T
    b1n = bn1.reshape(1, D)
    W2nt = Wn2.T
    b2n = bn2.reshape(1, D)

    P, Q = _tc_prep(h, At, Bt)
    zeros_tile = jnp.zeros((ZCH, D), f32)

    esum, radf = _sc_gather(NCHUNKS)(P, Q, cx, cy, cz, rc)
    radp = radf.reshape(N_EDGES // BE, BE // CHUNK, CHUNK)
    mf = (jnp.arange(BE)[:, None] % CHUNK
          == jnp.arange(CHUNK)[None, :]).astype(f32)
    ones_col = jnp.ones((CHUNK, 1), f32)
    edge_feat = _tc_edge(esum, radp, mf, ones_col, wr, b1e, W2t, b2e)
    aggp = _sc_scatter(NCHUNKS)(edge_feat, rc, zeros_tile)
    h_out = _tc_node(h, aggp, W1ht, W1at, b1n, W2nt, b2n)
    return (h_out, coord, edge_feat)


# speedup vs baseline: 1.2001x; 1.0175x over previous
import functools

import jax
import jax.numpy as jnp
from jax import lax
from jax.experimental import pallas as pl
from jax.experimental.pallas import tpu as pltpu
from jax.experimental.pallas import tpu_sc as plsc

N_NODES = 10000
N_EDGES = 320000
D = 128
CPAD = 4
CHUNK = 128
NCHUNKS = N_EDGES // CHUNK
NGRP = CHUNK // 16

NC = 2
NS = 16
NW = NC * NS

ZCH = 80
NZCH = N_NODES // ZCH


def _leaky(x):
    return jnp.where(x > 0, x, 0.2 * x)


UNROLL = 6


def _sc_gather_body(nch, p_hbm, q_hbm, cx_hbm, cy_hbm, cz_hbm, rc_hbm,
                    esum_hbm, rad_hbm,
                    idx_v, buf_p, buf_q, buf_rad, cx_t, cy_t, cz_t,
                    gsem0, gsem1, wsem0, wsem1, wsem2,
                    isem0, isem1, isem2):
    NLOOP = UNROLL * pl.cdiv(pl.cdiv(nch, NW), UNROLL)
    wid = lax.axis_index("s") * NC + lax.axis_index("c")
    gsems = (gsem0, gsem1)
    wsems = (wsem0, wsem1, wsem2)
    isems = (isem0, isem1, isem2)

    pltpu.sync_copy(cx_hbm, cx_t)
    pltpu.sync_copy(cy_hbm, cy_t)
    pltpu.sync_copy(cz_hbm, cz_t)

    def cid_of(i):
        return i * NW + wid

    def issue_idx(isl, cid):
        pltpu.async_copy(rc_hbm.at[cid], idx_v.at[isl], isems[isl])

    def wait_idx(isl):
        pltpu.make_async_copy(rc_hbm.at[0], idx_v.at[isl], isems[isl]).wait()

    def issue(isl, qs, ps):
        pltpu.async_copy(p_hbm.at[idx_v.at[isl, 0]], buf_p.at[ps], gsems[qs])
        pltpu.async_copy(q_hbm.at[idx_v.at[isl, 1]], buf_q.at[qs], gsems[qs])

    def wait_gathers(isl, qs, ps):
        pltpu.make_async_copy(p_hbm.at[idx_v.at[isl, 0]], buf_p.at[ps],
                              gsems[qs]).wait()
        pltpu.make_async_copy(q_hbm.at[idx_v.at[isl, 1]], buf_q.at[qs],
                              gsems[qs]).wait()

    def write(ps, cid):
        pltpu.async_copy(buf_p.at[ps], esum_hbm.at[pl.ds(cid * CHUNK, CHUNK)],
                         wsems[ps])
        pltpu.async_copy(buf_rad.at[ps], rad_hbm.at[pl.ds(cid * CHUNK, CHUNK)],
                         wsems[ps])

    def wait_write(ps):
        pltpu.make_async_copy(buf_p.at[ps], esum_hbm.at[pl.ds(0, CHUNK)],
                              wsems[ps]).wait()
        pltpu.make_async_copy(buf_rad.at[ps], rad_hbm.at[pl.ds(0, CHUNK)],
                              wsems[ps]).wait()

    def fuse(isl, qs, ps):
        def grp_body(g, carry):
            iv = idx_v[isl, 0, pl.ds(g * 16, 16)]
            jv = idx_v[isl, 1, pl.ds(g * 16, 16)]
            dx = plsc.load_gather(cx_t, [iv]) - plsc.load_gather(cx_t, [jv])
            dy = plsc.load_gather(cy_t, [iv]) - plsc.load_gather(cy_t, [jv])
            dz = plsc.load_gather(cz_t, [iv]) - plsc.load_gather(cz_t, [jv])
            buf_rad[ps, pl.ds(g * 16, 16)] = dx * dx + dy * dy + dz * dz
            return carry

        lax.fori_loop(0, NGRP, grp_body, 0)

        def add_body(e, carry):
            for k in range(D // 16):
                sl = pl.ds(k * 16, 16)
                plsc.addupdate(buf_p.at[ps, e, sl], buf_q[qs, e, sl])
            return carry

        lax.fori_loop(0, CHUNK, add_body, 0)

    issue_idx(0, cid_of(0))

    @pl.when(cid_of(1) < nch)
    def _():
        issue_idx(1, cid_of(1))

    wait_idx(0)
    issue(0, 0, 0)

    def outer(j, carry):
        for t in range(UNROLL):
            i = j * UNROLL + t
            isl, qs, ps = t % 3, t % 2, t % 3
            isl_n, qs_n, ps_n = (t + 1) % 3, (t + 1) % 2, (t + 1) % 3
            isl_n2 = (t + 2) % 3

            @pl.when(cid_of(i + 2) < nch)
            def _():
                issue_idx(isl_n2, cid_of(i + 2))

            @pl.when(cid_of(i + 1) < nch)
            def _():
                @pl.when(i >= 2)
                def _():
                    wait_write(ps_n)
                wait_idx(isl_n)
                issue(isl_n, qs_n, ps_n)

            @pl.when(cid_of(i) < nch)
            def _():
                wait_gathers(isl, qs, ps)
                fuse(isl, qs, ps)
                write(ps, cid_of(i))

        return carry

    lax.fori_loop(0, NLOOP // UNROLL, outer, 0)
    wait_write(0)
    wait_write(1)
    wait_write(2)


@functools.lru_cache(maxsize=None)
def _sc_gather(nch):
    return pl.kernel(
        functools.partial(_sc_gather_body, nch),
        out_type=(
            jax.ShapeDtypeStruct((nch * CHUNK, D), jnp.float32),
            jax.ShapeDtypeStruct((nch * CHUNK,), jnp.float32),
        ),
        mesh=plsc.VectorSubcoreMesh(
            core_axis_name="c", subcore_axis_name="s",
            num_cores=NC, num_subcores=NS),
        scratch_types=[
            pltpu.VMEM((3, 2, CHUNK), jnp.int32),
            pltpu.VMEM((3, CHUNK, D), jnp.float32),
            pltpu.VMEM((2, CHUNK, D), jnp.float32),
            pltpu.VMEM((3, CHUNK), jnp.float32),
            pltpu.VMEM((N_NODES,), jnp.float32),
            pltpu.VMEM((N_NODES,), jnp.float32),
            pltpu.VMEM((N_NODES,), jnp.float32),
            pltpu.SemaphoreType.DMA,
            pltpu.SemaphoreType.DMA,
            pltpu.SemaphoreType.DMA,
            pltpu.SemaphoreType.DMA,
            pltpu.SemaphoreType.DMA,
            pltpu.SemaphoreType.DMA,
            pltpu.SemaphoreType.DMA,
            pltpu.SemaphoreType.DMA,
        ],
        compiler_params=pltpu.CompilerParams(needs_layout_passes=False),
    )


def _sc_scatter_body(nch, ef_hbm, rc_hbm, zero_hbm, out_hbm,
                     idx_v, ef_v, agg_sh,
                     rsem0, rsem1, ssem0, ssem1):
    NSTEPS = 2 * pl.cdiv(pl.cdiv(pl.cdiv(nch, NC), NS), 2)
    c = lax.axis_index("c")
    s = lax.axis_index("s")
    rsems = (rsem0, rsem1)
    ssems = (ssem0, ssem1)

    z_v = ef_v.at[0, pl.ds(0, ZCH)]
    pltpu.sync_copy(zero_hbm, z_v)
    for i in range(pl.cdiv(NZCH, NS)):
        j = i * NS + s

        @pl.when(j < NZCH)
        def _():
            pltpu.sync_copy(z_v, agg_sh.at[pl.ds(j * ZCH, ZCH)])

    plsc.subcore_barrier()

    def jm_of(i):
        return i * NS + s

    def issue_read(b, m):
        pltpu.async_copy(rc_hbm.at[m], idx_v.at[b], rsems[b])
        pltpu.async_copy(ef_hbm.at[pl.ds(m * CHUNK, CHUNK)], ef_v.at[b],
                         rsems[b])

    def wait_read(b):
        pltpu.make_async_copy(rc_hbm.at[0], idx_v.at[b], rsems[b]).wait()
        pltpu.make_async_copy(ef_hbm.at[pl.ds(0, CHUNK)], ef_v.at[b],
                              rsems[b]).wait()

    def issue_scatter(b):
        pltpu.async_copy(ef_v.at[b], agg_sh.at[idx_v.at[b, 0]], ssems[b],
                         add=True)

    def wait_scatter(b):
        pltpu.make_async_copy(ef_v.at[b], agg_sh.at[idx_v.at[b, 0]],
                              ssems[b]).wait()

    issue_read(0, jm_of(0) * NC + c)

    def outer(jj, carry):
        for t in range(2):
            i = jj * 2 + t
            b, nb = t, 1 - t

            @pl.when(jm_of(i + 1) * NC + c < nch)
            def _():
                @pl.when(i >= 1)
                def _():
                    wait_scatter(nb)
                issue_read(nb, jm_of(i + 1) * NC + c)

            @pl.when(jm_of(i) * NC + c < nch)
            def _():
                wait_read(b)
                issue_scatter(b)

        return carry

    lax.fori_loop(0, NSTEPS // 2, outer, 0)
    wait_scatter(0)
    wait_scatter(1)
    plsc.subcore_barrier()

    for i in range(pl.cdiv(NZCH, NS)):
        j = i * NS + s

        @pl.when(j < NZCH)
        def _():
            off = j * ZCH
            pltpu.sync_copy(agg_sh.at[pl.ds(off, ZCH)], z_v)
            pltpu.sync_copy(z_v, out_hbm.at[c, pl.ds(off, ZCH)])


@functools.lru_cache(maxsize=None)
def _sc_scatter(nch):
    return pl.kernel(
        functools.partial(_sc_scatter_body, nch),
        out_type=jax.ShapeDtypeStruct((NC, N_NODES, D), jnp.float32),
        mesh=plsc.VectorSubcoreMesh(
            core_axis_name="c", subcore_axis_name="s",
            num_cores=NC, num_subcores=NS),
        scratch_types=[
            pltpu.VMEM((2, 2, CHUNK), jnp.int32),
            pltpu.VMEM((2, CHUNK, D), jnp.float32),
            pltpu.VMEM_SHARED((N_NODES, D), jnp.float32),
            pltpu.SemaphoreType.DMA,
            pltpu.SemaphoreType.DMA,
            pltpu.SemaphoreType.DMA,
            pltpu.SemaphoreType.DMA,
        ],
    )


def _tc_prep_body(h_ref, at_ref, bt_ref, p_ref, q_ref):
    hh = h_ref[...]
    p_ref[...] = jnp.dot(hh, at_ref[...], preferred_element_type=jnp.float32)
    q_ref[...] = jnp.dot(hh, bt_ref[...], preferred_element_type=jnp.float32)


def _tc_prep(h, At, Bt):
    return pl.pallas_call(
        _tc_prep_body,
        out_shape=(
            jax.ShapeDtypeStruct((N_NODES, D), jnp.float32),
            jax.ShapeDtypeStruct((N_NODES, D), jnp.float32),
        ),
    )(h, At, Bt)


BE = 3200


def _tc_edge_body(es_ref, rad_ref, mf_ref, ones_ref, wr_ref, b1_ref, w2_ref,
                  b2_ref, out_ref):
    radm = rad_ref[0]
    rows = jnp.broadcast_to(radm[:, None, :], (BE // CHUNK, CHUNK, CHUNK))
    rows = rows.reshape(BE, CHUNK)
    radial = jnp.dot(rows * mf_ref[...], ones_ref[...],
                     preferred_element_type=jnp.float32)
    x = _leaky(es_ref[...] + radial * wr_ref[...] + b1_ref[...])
    y = jnp.dot(x, w2_ref[...], preferred_element_type=jnp.float32) + b2_ref[...]
    out_ref[...] = _leaky(y)


def _tc_edge(esum, radp, mf, ones_col, wr, be1, W2t, be2):
    ne = esum.shape[0]
    grid = (ne // BE,)
    return pl.pallas_call(
        _tc_edge_body,
        grid=grid,
        in_specs=[
            pl.BlockSpec((BE, D), lambda i: (i, 0)),
            pl.BlockSpec((1, BE // CHUNK, CHUNK), lambda i: (i, 0, 0)),
            pl.BlockSpec((BE, CHUNK), lambda i: (0, 0)),
            pl.BlockSpec((CHUNK, 1), lambda i: (0, 0)),
            pl.BlockSpec((1, D), lambda i: (0, 0)),
            pl.BlockSpec((1, D), lambda i: (0, 0)),
            pl.BlockSpec((D, D), lambda i: (0, 0)),
            pl.BlockSpec((1, D), lambda i: (0, 0)),
        ],
        out_specs=pl.BlockSpec((BE, D), lambda i: (i, 0)),
        out_shape=jax.ShapeDtypeStruct((ne, D), jnp.float32),
    )(esum, radp, mf, ones_col, wr, be1, W2t, be2)


def _tc_node_body(h_ref, agg_ref, w1h_ref, w1a_ref, b1_ref,
                  w2_ref, b2_ref, out_ref):
    hh = h_ref[...]
    agg = agg_ref[0] + agg_ref[1]
    x = (jnp.dot(hh, w1h_ref[...], preferred_element_type=jnp.float32)
         + jnp.dot(agg, w1a_ref[...], preferred_element_type=jnp.float32)
         + b1_ref[...])
    x = _leaky(x)
    y = jnp.dot(x, w2_ref[...], preferred_element_type=jnp.float32) + b2_ref[...]
    out_ref[...] = hh + y


def _tc_node(h, aggp, W1ht, W1at, bn1, W2t, bn2):
    return pl.pallas_call(
        _tc_node_body,
        out_shape=jax.ShapeDtypeStruct((N_NODES, D), jnp.float32),
    )(h, aggp, W1ht, W1at, bn1, W2t, bn2)


def kernel(h, edge_index, coord, We1, be1, We2, be2, Wn1, bn1, Wn2, bn2):
    f32 = jnp.float32
    row = edge_index[0].astype(jnp.int32)
    col = edge_index[1].astype(jnp.int32)
    rc = jnp.stack([row.reshape(NCHUNKS, CHUNK),
                    col.reshape(NCHUNKS, CHUNK)], axis=1)
    cx = coord[:, 0].astype(f32)
    cy = coord[:, 1].astype(f32)
    cz = coord[:, 2].astype(f32)

    At = We1[:, :D].T
    Bt = We1[:, D:2 * D].T
    wr = We1[:, 2 * D].reshape(1, D)
    b1e = be1.reshape(1, D)
    W2t = We2.T
    b2e = be2.reshape(1, D)
    W1ht = Wn1[:, :D].T
    W1at = Wn1[:, D:].T
    b1n = bn1.reshape(1, D)
    W2nt = Wn2.T
    b2n = bn2.reshape(1, D)

    P, Q = _tc_prep(h, At, Bt)
    zeros_tile = jnp.zeros((ZCH, D), f32)

    esum, radf = _sc_gather(NCHUNKS)(P, Q, cx, cy, cz, rc)
    radp = radf.reshape(N_EDGES // BE, BE // CHUNK, CHUNK)
    mf = (jnp.arange(BE)[:, None] % CHUNK
          == jnp.arange(CHUNK)[None, :]).astype(f32)
    ones_col = jnp.ones((CHUNK, 1), f32)
    edge_feat = _tc_edge(esum, radp, mf, ones_col, wr, b1e, W2t, b2e)
    aggp = _sc_scatter(NCHUNKS)(edge_feat, rc, zeros_tile)
    h_out = _tc_node(h, aggp, W1ht, W1at, b1n, W2nt, b2n)
    return (h_out, coord, edge_feat)
